# baseline (speedup 1.0000x reference)
import functools
import math

import jax
import jax.numpy as jnp
import numpy as np
from jax import lax
from jax.experimental import pallas as pl
from jax.experimental.pallas import tpu as pltpu
from jax.experimental.pallas import tpu_sc as plsc

N = 10000
E = 160000
D = 16
HID = 256

NC = 2
NS = 16
NW = NC * NS
CH = 128
NCH = 40
PER_W = NCH * CH
E_PAD = NW * PER_W
ZR = N // NS


def _sc_gather_body(nf_hbm, src_hbm, out_hbm, idx_v, rows_v, sem):
    c = lax.axis_index("c")
    s = lax.axis_index("s")
    wid = s * NC + c
    pltpu.sync_copy(src_hbm.at[wid], idx_v)
    for g in range(0, NCH, 8):
        cps = [
            pltpu.async_copy(nf_hbm.at[idx_v.at[g + b]], rows_v.at[g + b], sem)
            for b in range(8)
        ]
        for cp in cps:
            cp.wait()
    pltpu.sync_copy(rows_v, out_hbm.at[wid])


def _sc_scatter_body(ef_hbm, dst_hbm, zero_hbm, part_hbm, idx_v, ef_v, acc, sem):
    c = lax.axis_index("c")
    s = lax.axis_index("s")
    wid = s * NC + c
    pltpu.sync_copy(zero_hbm.at[pl.ds(s * ZR, ZR)], acc.at[pl.ds(s * ZR, ZR)])
    pltpu.sync_copy(dst_hbm.at[wid], idx_v)
    pltpu.sync_copy(ef_hbm.at[wid], ef_v)
    plsc.subcore_barrier()
    for g in range(0, NCH, 8):
        cps = [
            pltpu.async_copy(ef_v.at[g + b], acc.at[idx_v.at[g + b]], sem, add=True)
            for b in range(8)
        ]
        for cp in cps:
            cp.wait()
    plsc.subcore_barrier()
    pltpu.sync_copy(acc.at[pl.ds(s * ZR, ZR)], part_hbm.at[c, pl.ds(s * ZR, ZR)])


_BLK = 2048
_INV_SQRT3 = 1.0 / math.sqrt(3.0)


def _split(v):
    hi = v.astype(jnp.bfloat16)
    lo = (v - hi.astype(jnp.float32)).astype(jnp.bfloat16)
    return hi, lo


def _tc_edge_body(
    sat_ref, x_ref, w1h_ref, w1l_ref, w2_ref, rmat_ref, smat_ref, out_ref
):
    sal = jnp.transpose(sat_ref[...])
    sal_h, sal_l = _split(sal)
    z = (
        jnp.dot(sal_h, w1h_ref[...], preferred_element_type=jnp.float32)
        + jnp.dot(sal_h, w1l_ref[...], preferred_element_type=jnp.float32)
        + jnp.dot(sal_l, w1h_ref[...], preferred_element_type=jnp.float32)
    )
    h = jnp.maximum(z, 0.0)
    w = jnp.dot(
        h.astype(jnp.bfloat16), w2_ref[...], preferred_element_type=jnp.float32
    )
    w = w * sal[:, 3:4]
    w3 = w.reshape(_BLK // 8, 8, HID)
    xp = x_ref[...]
    efs = []
    for m in range(8):
        xs = xp[:, D * m : D * (m + 1)]
        xh, xl = _split(xs)
        xr = jnp.dot(xh, rmat_ref[...], preferred_element_type=jnp.float32) + jnp.dot(
            xl, rmat_ref[...], preferred_element_type=jnp.float32
        )
        ph, plo = _split(xr * w3[:, m, :])
        efs.append(
            jnp.dot(ph, smat_ref[...], preferred_element_type=jnp.float32)
            + jnp.dot(plo, smat_ref[...], preferred_element_type=jnp.float32)
        )
    out_ref[...] = jnp.concatenate(efs, axis=1)


def _tc_edge(sat, x_packed, w1h, w1l, w2, rmat, smat):
    return pl.pallas_call(
        _tc_edge_body,
        grid=(E_PAD // _BLK,),
        in_specs=[
            pl.BlockSpec((4, _BLK), lambda i: (0, i)),
            pl.BlockSpec((_BLK // 8, 128), lambda i: (i, 0)),
            pl.BlockSpec((4, HID), lambda i: (0, 0)),
            pl.BlockSpec((4, HID), lambda i: (0, 0)),
            pl.BlockSpec((HID, HID), lambda i: (0, 0)),
            pl.BlockSpec((D, HID), lambda i: (0, 0)),
            pl.BlockSpec((HID, D), lambda i: (0, 0)),
        ],
        out_specs=pl.BlockSpec((_BLK // 8, 128), lambda i: (i, 0)),
        out_shape=jax.ShapeDtypeStruct((E_PAD // 8, 128), jnp.float32),
    )(sat, x_packed, w1h, w1l, w2, rmat, smat)


def _tc_combine_body(p_ref, o_ref):
    o_ref[...] = p_ref[0] + p_ref[1]


_PROWS = N * D // 128


def _tc_combine(parts_lin):
    return pl.pallas_call(
        _tc_combine_body,
        out_shape=jax.ShapeDtypeStruct((_PROWS, 128), jnp.float32),
    )(parts_lin)


@functools.cache
def _sc_kernels():
    mesh = plsc.VectorSubcoreMesh(core_axis_name="c", subcore_axis_name="s")
    gather = pl.kernel(
        _sc_gather_body,
        out_type=jax.ShapeDtypeStruct((NW, NCH, CH, D), jnp.float32),
        mesh=mesh,
        scratch_types=[
            pltpu.VMEM((NCH, CH), jnp.int32),
            pltpu.VMEM((NCH, CH, D), jnp.float32),
            pltpu.SemaphoreType.DMA,
        ],
        compiler_params=pltpu.CompilerParams(use_tc_tiling_on_sc=False),
    )
    scatter = pl.kernel(
        _sc_scatter_body,
        out_type=jax.ShapeDtypeStruct((NC, N, D), jnp.float32),
        mesh=mesh,
        scratch_types=[
            pltpu.VMEM((NCH, CH), jnp.int32),
            pltpu.VMEM((NCH, CH, D), jnp.float32),
            pltpu.VMEM_SHARED((N, D), jnp.float32),
            pltpu.SemaphoreType.DMA,
        ],
        compiler_params=pltpu.CompilerParams(use_tc_tiling_on_sc=False),
    )
    return gather, scatter


def kernel(node_features, edge_src, edge_dst, edge_attr, edge_scalars, W1, W2):
    _sc_gather, _sc_scatter = _sc_kernels()
    pad = E_PAD - E
    src = jnp.pad(edge_src, (0, pad)).reshape(NW, NCH, CH)
    dst = jnp.pad(edge_dst, (0, pad)).reshape(NW, NCH, CH)
    sat = jnp.pad(
        jnp.concatenate([edge_scalars.T, edge_attr.T], axis=0), ((0, 0), (0, pad))
    )

    w1s = jnp.pad(W1 * (_INV_SQRT3 / 256.0), ((0, 1), (0, 0)))
    w1h = w1s.astype(jnp.bfloat16)
    w1l = (w1s - w1h.astype(jnp.float32)).astype(jnp.bfloat16)
    w2b = W2.astype(jnp.bfloat16)
    i16 = np.arange(D)
    k256 = np.arange(HID)
    rmat = (k256[None, :] // D == i16[:, None]).astype(jnp.bfloat16)
    smat = (k256[:, None] % D == i16[None, :]).astype(jnp.bfloat16)

    x_packed = _sc_gather(node_features, src).reshape(E_PAD // 8, 128)
    ef = _tc_edge(sat, x_packed, w1h, w1l, w2b, rmat, smat).reshape(NW, NCH, CH, D)
    zeros = jnp.zeros((N, D), jnp.float32)
    parts = _sc_scatter(ef, dst, zeros)
    return _tc_combine(parts.reshape(NC, _PROWS, 128)).reshape(N, D)


# speedup vs baseline: 1.7919x; 1.7919x over previous
import functools
import math

import jax
import jax.numpy as jnp
import numpy as np
from jax import lax
from jax.experimental import pallas as pl
from jax.experimental.pallas import tpu as pltpu
from jax.experimental.pallas import tpu_sc as plsc

N = 10000
E = 160000
D = 16
HID = 256

NC = 2
NS = 16
NW = NC * NS
CH = 128
NCH = 40
PER_W = NCH * CH
E_PAD = NW * PER_W
ZR = N // NS


def _sc_gather_body(nf_hbm, src_hbm, out_hbm, idx_v, rows_v, sem):
    c = lax.axis_index("c")
    s = lax.axis_index("s")
    wid = s * NC + c
    pltpu.sync_copy(src_hbm.at[wid], idx_v)
    for g in range(0, NCH, 8):
        cps = [
            pltpu.async_copy(nf_hbm.at[idx_v.at[g + b]], rows_v.at[g + b], sem)
            for b in range(8)
        ]
        for cp in cps:
            cp.wait()
    pltpu.sync_copy(rows_v, out_hbm.at[wid])


def _sc_scatter_body(ef_hbm, dst_hbm, zero_hbm, part_hbm, idx_v, ef_v, acc, sem):
    c = lax.axis_index("c")
    s = lax.axis_index("s")
    wid = s * NC + c
    pltpu.sync_copy(zero_hbm.at[pl.ds(s * ZR, ZR)], acc.at[pl.ds(s * ZR, ZR)])
    pltpu.sync_copy(dst_hbm.at[wid], idx_v)
    pltpu.sync_copy(ef_hbm.at[wid], ef_v)
    plsc.subcore_barrier()
    for g in range(0, NCH, 8):
        cps = [
            pltpu.async_copy(ef_v.at[g + b], acc.at[idx_v.at[g + b]], sem, add=True)
            for b in range(8)
        ]
        for cp in cps:
            cp.wait()
    plsc.subcore_barrier()
    pltpu.sync_copy(acc.at[pl.ds(s * ZR, ZR)], part_hbm.at[c, pl.ds(s * ZR, ZR)])


_BLK = 2048
_INV_SQRT3 = 1.0 / math.sqrt(3.0)


def _tc_edge_body(sat_ref, x_ref, apk_ref, w1t_ref, w2_ref, arep_ref, rbig_ref, sbig_ref, out_ref):
    z_t = jnp.dot(w1t_ref[...], sat_ref[...], preferred_element_type=jnp.float32)
    h_t = jnp.maximum(z_t, 0.0).astype(jnp.bfloat16)
    w = lax.dot_general(
        h_t, w2_ref[...], (((0,), (0,)), ((), ())),
        preferred_element_type=jnp.float32,
    )
    wre = w.reshape(_BLK // 8, 8 * HID)
    attr_rep = jnp.dot(apk_ref[...], arep_ref[...], preferred_element_type=jnp.float32)
    xa = x_ref[...] * attr_rep
    xr = jnp.dot(xa, rbig_ref[...], preferred_element_type=jnp.float32)
    out_ref[...] = jnp.dot(xr * wre, sbig_ref[...], preferred_element_type=jnp.float32)


def _tc_edge(sat, x_packed, apk, w1t, w2, arep, rbig, sbig):
    return pl.pallas_call(
        _tc_edge_body,
        grid=(E_PAD // _BLK,),
        in_specs=[
            pl.BlockSpec((3, _BLK), lambda i: (0, i)),
            pl.BlockSpec((_BLK // 8, 128), lambda i: (i, 0)),
            pl.BlockSpec((_BLK // 8, 8), lambda i: (i, 0)),
            pl.BlockSpec((HID, 3), lambda i: (0, 0)),
            pl.BlockSpec((HID, HID), lambda i: (0, 0)),
            pl.BlockSpec((8, 128), lambda i: (0, 0)),
            pl.BlockSpec((128, 8 * HID), lambda i: (0, 0)),
            pl.BlockSpec((8 * HID, 128), lambda i: (0, 0)),
        ],
        out_specs=pl.BlockSpec((_BLK // 8, 128), lambda i: (i, 0)),
        out_shape=jax.ShapeDtypeStruct((E_PAD // 8, 128), jnp.float32),
    )(sat, x_packed, apk, w1t, w2, arep, rbig, sbig)


def _tc_combine_body(p_ref, o_ref):
    o_ref[...] = p_ref[0] + p_ref[1]


_PROWS = N * D // 128


def _tc_combine(parts_lin):
    return pl.pallas_call(
        _tc_combine_body,
        out_shape=jax.ShapeDtypeStruct((_PROWS, 128), jnp.float32),
    )(parts_lin)


@functools.cache
def _sc_kernels():
    mesh = plsc.VectorSubcoreMesh(core_axis_name="c", subcore_axis_name="s")
    gather = pl.kernel(
        _sc_gather_body,
        out_type=jax.ShapeDtypeStruct((NW, NCH, CH, D), jnp.float32),
        mesh=mesh,
        scratch_types=[
            pltpu.VMEM((NCH, CH), jnp.int32),
            pltpu.VMEM((NCH, CH, D), jnp.float32),
            pltpu.SemaphoreType.DMA,
        ],
        compiler_params=pltpu.CompilerParams(use_tc_tiling_on_sc=False),
    )
    scatter = pl.kernel(
        _sc_scatter_body,
        out_type=jax.ShapeDtypeStruct((NC, N, D), jnp.float32),
        mesh=mesh,
        scratch_types=[
            pltpu.VMEM((NCH, CH), jnp.int32),
            pltpu.VMEM((NCH, CH, D), jnp.float32),
            pltpu.VMEM_SHARED((N, D), jnp.float32),
            pltpu.SemaphoreType.DMA,
        ],
        compiler_params=pltpu.CompilerParams(use_tc_tiling_on_sc=False),
    )
    return gather, scatter


def kernel(node_features, edge_src, edge_dst, edge_attr, edge_scalars, W1, W2):
    _sc_gather, _sc_scatter = _sc_kernels()
    pad = E_PAD - E
    src = jnp.pad(edge_src, (0, pad)).reshape(NW, NCH, CH)
    dst = jnp.pad(edge_dst, (0, pad)).reshape(NW, NCH, CH)
    sat = jnp.pad(edge_scalars.T, ((0, 0), (0, pad)))
    apk = jnp.pad(edge_attr[:, 0], (0, pad)).reshape(E_PAD // 8, 8)

    w1t = (W1 * (_INV_SQRT3 / 256.0)).T
    w2b = W2.astype(jnp.bfloat16)
    m8 = np.arange(8)
    l128 = np.arange(128)
    c2048 = np.arange(8 * HID)
    arep = (m8[:, None] == l128[None, :] // D).astype(np.float32)
    rbig = (l128[:, None] == (c2048[None, :] // HID) * D + (c2048[None, :] % HID) // D
            ).astype(np.float32)
    sbig = ((c2048[:, None] // HID == l128[None, :] // D)
            & (c2048[:, None] % D == l128[None, :] % D)).astype(np.float32)

    x_packed = _sc_gather(node_features, src).reshape(E_PAD // 8, 128)
    ef = _tc_edge(sat, x_packed, apk, w1t, w2b, arep, rbig, sbig).reshape(
        NW, NCH, CH, D
    )
    zeros = jnp.zeros((N, D), jnp.float32)
    parts = _sc_scatter(ef, dst, zeros)
    return _tc_combine(parts.reshape(NC, _PROWS, 128)).reshape(N, D)


# speedup vs baseline: 1.9790x; 1.1044x over previous
import functools
import math

import jax
import jax.numpy as jnp
import numpy as np
from jax import lax
from jax.experimental import pallas as pl
from jax.experimental.pallas import tpu as pltpu
from jax.experimental.pallas import tpu_sc as plsc

N = 10000
E = 160000
D = 16
HID = 256

NC = 2
NS = 16
NW = NC * NS
CH = 128
NCH = 40
PER_W = NCH * CH
E_PAD = NW * PER_W
ZR = N // NS


def _sc_gather_body(nf_hbm, src_hbm, out_hbm, idx_v, rows_v, sem):
    c = lax.axis_index("c")
    s = lax.axis_index("s")
    wid = s * NC + c
    pltpu.sync_copy(src_hbm.at[wid], idx_v)
    for g in range(0, NCH, 8):
        cps = [
            pltpu.async_copy(nf_hbm.at[idx_v.at[g + b]], rows_v.at[g + b], sem)
            for b in range(8)
        ]
        for cp in cps:
            cp.wait()
    pltpu.sync_copy(rows_v, out_hbm.at[wid])


def _sc_scatter_body(ef_hbm, dst_hbm, zero_hbm, part_hbm, idx_v, ef_v, acc, sem):
    c = lax.axis_index("c")
    s = lax.axis_index("s")
    wid = s * NC + c
    pltpu.sync_copy(zero_hbm.at[pl.ds(s * ZR, ZR)], acc.at[pl.ds(s * ZR, ZR)])
    pltpu.sync_copy(dst_hbm.at[wid], idx_v)
    pltpu.sync_copy(ef_hbm.at[wid], ef_v)
    plsc.subcore_barrier()
    for g in range(0, NCH, 8):
        cps = [
            pltpu.async_copy(ef_v.at[g + b], acc.at[idx_v.at[g + b]], sem, add=True)
            for b in range(8)
        ]
        for cp in cps:
            cp.wait()
    plsc.subcore_barrier()
    pltpu.sync_copy(acc.at[pl.ds(s * ZR, ZR)], part_hbm.at[c, pl.ds(s * ZR, ZR)])


_BLK = 2048
_INV_SQRT3 = 1.0 / math.sqrt(3.0)


def _tc_edge_body(sat_ref, x_ref, apk_ref, w1t_ref, w2_ref, arep_ref, rbig_ref, sbig_ref, out_ref):
    z_t = jnp.dot(
        w1t_ref[...], sat_ref[...].astype(jnp.bfloat16),
        preferred_element_type=jnp.float32,
    )
    h_t = jnp.maximum(z_t, 0.0).astype(jnp.bfloat16)
    w = lax.dot_general(
        h_t, w2_ref[...], (((0,), (0,)), ((), ())),
        preferred_element_type=jnp.float32,
    )
    wre = w.reshape(_BLK // 8, 8 * HID)
    attr_rep = jnp.dot(apk_ref[...], arep_ref[...], preferred_element_type=jnp.float32)
    xa = (x_ref[...] * attr_rep).astype(jnp.bfloat16)
    xr = jnp.dot(xa, rbig_ref[...], preferred_element_type=jnp.float32)
    prod = (xr * wre).astype(jnp.bfloat16)
    out_ref[...] = jnp.dot(prod, sbig_ref[...], preferred_element_type=jnp.float32)


def _tc_edge(sat, x_packed, apk, w1t, w2, arep, rbig, sbig):
    return pl.pallas_call(
        _tc_edge_body,
        grid=(E_PAD // _BLK,),
        in_specs=[
            pl.BlockSpec((3, _BLK), lambda i: (0, i)),
            pl.BlockSpec((_BLK // 8, 128), lambda i: (i, 0)),
            pl.BlockSpec((_BLK // 8, 8), lambda i: (i, 0)),
            pl.BlockSpec((HID, 3), lambda i: (0, 0)),
            pl.BlockSpec((HID, HID), lambda i: (0, 0)),
            pl.BlockSpec((8, 128), lambda i: (0, 0)),
            pl.BlockSpec((128, 8 * HID), lambda i: (0, 0)),
            pl.BlockSpec((8 * HID, 128), lambda i: (0, 0)),
        ],
        out_specs=pl.BlockSpec((_BLK // 8, 128), lambda i: (i, 0)),
        out_shape=jax.ShapeDtypeStruct((E_PAD // 8, 128), jnp.float32),
    )(sat, x_packed, apk, w1t, w2, arep, rbig, sbig)


def _tc_combine_body(p_ref, o_ref):
    o_ref[...] = p_ref[0] + p_ref[1]


_PROWS = N * D // 128


def _tc_combine(parts_lin):
    return pl.pallas_call(
        _tc_combine_body,
        out_shape=jax.ShapeDtypeStruct((_PROWS, 128), jnp.float32),
    )(parts_lin)


@functools.cache
def _sc_kernels():
    mesh = plsc.VectorSubcoreMesh(core_axis_name="c", subcore_axis_name="s")
    gather = pl.kernel(
        _sc_gather_body,
        out_type=jax.ShapeDtypeStruct((NW, NCH, CH, D), jnp.float32),
        mesh=mesh,
        scratch_types=[
            pltpu.VMEM((NCH, CH), jnp.int32),
            pltpu.VMEM((NCH, CH, D), jnp.float32),
            pltpu.SemaphoreType.DMA,
        ],
        compiler_params=pltpu.CompilerParams(use_tc_tiling_on_sc=False),
    )
    scatter = pl.kernel(
        _sc_scatter_body,
        out_type=jax.ShapeDtypeStruct((NC, N, D), jnp.float32),
        mesh=mesh,
        scratch_types=[
            pltpu.VMEM((NCH, CH), jnp.int32),
            pltpu.VMEM((NCH, CH, D), jnp.float32),
            pltpu.VMEM_SHARED((N, D), jnp.float32),
            pltpu.SemaphoreType.DMA,
        ],
        compiler_params=pltpu.CompilerParams(use_tc_tiling_on_sc=False),
    )
    return gather, scatter


def kernel(node_features, edge_src, edge_dst, edge_attr, edge_scalars, W1, W2):
    _sc_gather, _sc_scatter = _sc_kernels()
    pad = E_PAD - E
    src = jnp.pad(edge_src, (0, pad)).reshape(NW, NCH, CH)
    dst = jnp.pad(edge_dst, (0, pad)).reshape(NW, NCH, CH)
    sat = jnp.pad(edge_scalars.T, ((0, 0), (0, pad)))
    apk = jnp.pad(edge_attr[:, 0], (0, pad)).reshape(E_PAD // 8, 8)

    w1t = (W1 * (_INV_SQRT3 / 256.0)).T.astype(jnp.bfloat16)
    w2b = W2.astype(jnp.bfloat16)
    m8 = np.arange(8)
    l128 = np.arange(128)
    c2048 = np.arange(8 * HID)
    arep = (m8[:, None] == l128[None, :] // D).astype(np.float32)
    rbig = (l128[:, None] == (c2048[None, :] // HID) * D + (c2048[None, :] % HID) // D
            ).astype(jnp.bfloat16)
    sbig = ((c2048[:, None] // HID == l128[None, :] // D)
            & (c2048[:, None] % D == l128[None, :] % D)).astype(jnp.bfloat16)

    x_packed = _sc_gather(node_features, src).reshape(E_PAD // 8, 128)
    ef = _tc_edge(sat, x_packed, apk, w1t, w2b, arep, rbig, sbig).reshape(
        NW, NCH, CH, D
    )
    zeros = jnp.zeros((N, D), jnp.float32)
    parts = _sc_scatter(ef, dst, zeros)
    return _tc_combine(parts.reshape(NC, _PROWS, 128)).reshape(N, D)


# speedup vs baseline: 2.0345x; 1.0280x over previous
import functools
import math

import jax
import jax.numpy as jnp
import numpy as np
from jax import lax
from jax.experimental import pallas as pl
from jax.experimental.pallas import tpu as pltpu
from jax.experimental.pallas import tpu_sc as plsc

N = 10000
E = 160000
D = 16
HID = 256

NC = 2
NS = 16
NW = NC * NS
CH = 128
NCH = 40
PER_W = NCH * CH
E_PAD = NW * PER_W
ZR = N // NS


def _sc_gather_body(nf_hbm, src_hbm, out_hbm, idx_v, rows_v, sem):
    c = lax.axis_index("c")
    s = lax.axis_index("s")
    wid = s * NC + c
    pltpu.sync_copy(src_hbm.at[wid], idx_v)
    for g in range(0, NCH, 8):
        cps = [
            pltpu.async_copy(nf_hbm.at[idx_v.at[g + b]], rows_v.at[g + b], sem)
            for b in range(8)
        ]
        for cp in cps:
            cp.wait()
    pltpu.sync_copy(rows_v, out_hbm.at[wid])


def _sc_scatter_body(ef_hbm, dst_hbm, zero_hbm, part_hbm, idx_v, ef_v, acc, sem):
    c = lax.axis_index("c")
    s = lax.axis_index("s")
    wid = s * NC + c
    pltpu.sync_copy(zero_hbm.at[pl.ds(s * ZR, ZR)], acc.at[pl.ds(s * ZR, ZR)])
    pltpu.sync_copy(dst_hbm.at[wid], idx_v)
    pltpu.sync_copy(ef_hbm.at[wid], ef_v)
    plsc.subcore_barrier()
    for g in range(0, NCH, 8):
        cps = [
            pltpu.async_copy(ef_v.at[g + b], acc.at[idx_v.at[g + b]], sem, add=True)
            for b in range(8)
        ]
        for cp in cps:
            cp.wait()
    plsc.subcore_barrier()
    pltpu.sync_copy(acc.at[pl.ds(s * ZR, ZR)], part_hbm.at[c, pl.ds(s * ZR, ZR)])


_BLK = 4096
_INV_SQRT3 = 1.0 / math.sqrt(3.0)


def _tc_edge_body(sat_ref, x_ref, apk_ref, w1t_ref, w2_ref, arep_ref, rbig_ref, sbig_ref, out_ref):
    z_t = jnp.dot(
        w1t_ref[...], sat_ref[...].astype(jnp.bfloat16),
        preferred_element_type=jnp.float32,
    )
    h_t = jnp.maximum(z_t, 0.0).astype(jnp.bfloat16)
    w = lax.dot_general(
        h_t, w2_ref[...], (((0,), (0,)), ((), ())),
        preferred_element_type=jnp.float32,
    )
    wre = w.reshape(_BLK // 8, 8 * HID)
    attr_rep = jnp.dot(apk_ref[...], arep_ref[...], preferred_element_type=jnp.float32)
    xa = (x_ref[...] * attr_rep).astype(jnp.bfloat16)
    xr = jnp.dot(xa, rbig_ref[...], preferred_element_type=jnp.float32)
    prod = (xr * wre).astype(jnp.bfloat16)
    out_ref[...] = jnp.dot(prod, sbig_ref[...], preferred_element_type=jnp.float32)


def _tc_edge(sat, x_packed, apk, w1t, w2, arep, rbig, sbig):
    return pl.pallas_call(
        _tc_edge_body,
        grid=(E_PAD // _BLK,),
        in_specs=[
            pl.BlockSpec((3, _BLK), lambda i: (0, i)),
            pl.BlockSpec((_BLK // 8, 128), lambda i: (i, 0)),
            pl.BlockSpec((_BLK // 8, 8), lambda i: (i, 0)),
            pl.BlockSpec((HID, 3), lambda i: (0, 0)),
            pl.BlockSpec((HID, HID), lambda i: (0, 0)),
            pl.BlockSpec((8, 128), lambda i: (0, 0)),
            pl.BlockSpec((128, 8 * HID), lambda i: (0, 0)),
            pl.BlockSpec((8 * HID, 128), lambda i: (0, 0)),
        ],
        out_specs=pl.BlockSpec((_BLK // 8, 128), lambda i: (i, 0)),
        out_shape=jax.ShapeDtypeStruct((E_PAD // 8, 128), jnp.float32),
    )(sat, x_packed, apk, w1t, w2, arep, rbig, sbig)


def _tc_combine_body(p_ref, o_ref):
    o_ref[...] = p_ref[0] + p_ref[1]


_PROWS = N * D // 128


def _tc_combine(parts_lin):
    return pl.pallas_call(
        _tc_combine_body,
        out_shape=jax.ShapeDtypeStruct((_PROWS, 128), jnp.float32),
    )(parts_lin)


@functools.cache
def _sc_kernels():
    mesh = plsc.VectorSubcoreMesh(core_axis_name="c", subcore_axis_name="s")
    gather = pl.kernel(
        _sc_gather_body,
        out_type=jax.ShapeDtypeStruct((NW, NCH, CH, D), jnp.float32),
        mesh=mesh,
        scratch_types=[
            pltpu.VMEM((NCH, CH), jnp.int32),
            pltpu.VMEM((NCH, CH, D), jnp.float32),
            pltpu.SemaphoreType.DMA,
        ],
        compiler_params=pltpu.CompilerParams(use_tc_tiling_on_sc=False),
    )
    scatter = pl.kernel(
        _sc_scatter_body,
        out_type=jax.ShapeDtypeStruct((NC, N, D), jnp.float32),
        mesh=mesh,
        scratch_types=[
            pltpu.VMEM((NCH, CH), jnp.int32),
            pltpu.VMEM((NCH, CH, D), jnp.float32),
            pltpu.VMEM_SHARED((N, D), jnp.float32),
            pltpu.SemaphoreType.DMA,
        ],
        compiler_params=pltpu.CompilerParams(use_tc_tiling_on_sc=False),
    )
    return gather, scatter


def kernel(node_features, edge_src, edge_dst, edge_attr, edge_scalars, W1, W2):
    _sc_gather, _sc_scatter = _sc_kernels()
    pad = E_PAD - E
    src = jnp.pad(edge_src, (0, pad)).reshape(NW, NCH, CH)
    dst = jnp.pad(edge_dst, (0, pad)).reshape(NW, NCH, CH)
    sat = jnp.pad(edge_scalars.T, ((0, 0), (0, pad)))
    apk = jnp.pad(edge_attr[:, 0], (0, pad)).reshape(E_PAD // 8, 8)

    w1t = (W1 * (_INV_SQRT3 / 256.0)).T.astype(jnp.bfloat16)
    w2b = W2.astype(jnp.bfloat16)
    m8 = np.arange(8)
    l128 = np.arange(128)
    c2048 = np.arange(8 * HID)
    arep = (m8[:, None] == l128[None, :] // D).astype(np.float32)
    rbig = (l128[:, None] == (c2048[None, :] // HID) * D + (c2048[None, :] % HID) // D
            ).astype(jnp.bfloat16)
    sbig = ((c2048[:, None] // HID == l128[None, :] // D)
            & (c2048[:, None] % D == l128[None, :] % D)).astype(jnp.bfloat16)

    x_packed = _sc_gather(node_features, src).reshape(E_PAD // 8, 128)
    ef = _tc_edge(sat, x_packed, apk, w1t, w2b, arep, rbig, sbig).reshape(
        NW, NCH, CH, D
    )
    zeros = jnp.zeros((N, D), jnp.float32)
    parts = _sc_scatter(ef, dst, zeros)
    return _tc_combine(parts.reshape(NC, _PROWS, 128)).reshape(N, D)


# speedup vs baseline: 2.0989x; 1.0317x over previous
import functools
import math

import jax
import jax.numpy as jnp
import numpy as np
from jax import lax
from jax.experimental import pallas as pl
from jax.experimental.pallas import tpu as pltpu
from jax.experimental.pallas import tpu_sc as plsc

N = 10000
E = 160000
D = 16
HID = 256

NC = 2
NS = 16
NW = NC * NS
CH = 128
NCH = 40
PER_W = NCH * CH
E_PAD = NW * PER_W
ZR = N // NS


def _sc_gather_body(nf_hbm, src_hbm, out_hbm, idx_v, rows_v, sem):
    c = lax.axis_index("c")
    s = lax.axis_index("s")
    wid = s * NC + c
    pltpu.sync_copy(src_hbm.at[wid], idx_v)
    for g in range(0, NCH, 8):
        cps = [
            pltpu.async_copy(nf_hbm.at[idx_v.at[g + b]], rows_v.at[g + b], sem)
            for b in range(8)
        ]
        for cp in cps:
            cp.wait()
    pltpu.sync_copy(rows_v, out_hbm.at[wid])


def _sc_scatter_body(ef_hbm, dst_hbm, zero_hbm, part_hbm, idx_v, ef_v, acc, sem):
    c = lax.axis_index("c")
    s = lax.axis_index("s")
    wid = s * NC + c
    pltpu.sync_copy(zero_hbm.at[pl.ds(s * ZR, ZR)], acc.at[pl.ds(s * ZR, ZR)])
    pltpu.sync_copy(dst_hbm.at[wid], idx_v)
    pltpu.sync_copy(ef_hbm.at[wid], ef_v)
    plsc.subcore_barrier()
    for g in range(0, NCH, 8):
        cps = [
            pltpu.async_copy(ef_v.at[g + b], acc.at[idx_v.at[g + b]], sem, add=True)
            for b in range(8)
        ]
        for cp in cps:
            cp.wait()
    plsc.subcore_barrier()
    pltpu.sync_copy(acc.at[pl.ds(s * ZR, ZR)], part_hbm.at[c, pl.ds(s * ZR, ZR)])


_BLK = 8192
_INV_SQRT3 = 1.0 / math.sqrt(3.0)


def _tc_edge_body(sat_ref, x_ref, apk_ref, w1t_ref, w2_ref, arep_ref, rbig_ref, sbig_ref, out_ref):
    z_t = jnp.dot(
        w1t_ref[...], sat_ref[...].astype(jnp.bfloat16),
        preferred_element_type=jnp.float32,
    )
    h_t = jnp.maximum(z_t, 0.0).astype(jnp.bfloat16)
    w = lax.dot_general(
        h_t, w2_ref[...], (((0,), (0,)), ((), ())),
        preferred_element_type=jnp.float32,
    )
    wre = w.reshape(_BLK // 8, 8 * HID)
    attr_rep = jnp.dot(apk_ref[...], arep_ref[...], preferred_element_type=jnp.float32)
    xa = (x_ref[...] * attr_rep).astype(jnp.bfloat16)
    xr = jnp.dot(xa, rbig_ref[...], preferred_element_type=jnp.float32)
    prod = (xr * wre).astype(jnp.bfloat16)
    out_ref[...] = jnp.dot(prod, sbig_ref[...], preferred_element_type=jnp.float32)


def _tc_edge(sat, x_packed, apk, w1t, w2, arep, rbig, sbig):
    return pl.pallas_call(
        _tc_edge_body,
        grid=(E_PAD // _BLK,),
        in_specs=[
            pl.BlockSpec((3, _BLK), lambda i: (0, i)),
            pl.BlockSpec((_BLK // 8, 128), lambda i: (i, 0)),
            pl.BlockSpec((_BLK // 8, 8), lambda i: (i, 0)),
            pl.BlockSpec((HID, 3), lambda i: (0, 0)),
            pl.BlockSpec((HID, HID), lambda i: (0, 0)),
            pl.BlockSpec((8, 128), lambda i: (0, 0)),
            pl.BlockSpec((128, 8 * HID), lambda i: (0, 0)),
            pl.BlockSpec((8 * HID, 128), lambda i: (0, 0)),
        ],
        out_specs=pl.BlockSpec((_BLK // 8, 128), lambda i: (i, 0)),
        out_shape=jax.ShapeDtypeStruct((E_PAD // 8, 128), jnp.float32),
    )(sat, x_packed, apk, w1t, w2, arep, rbig, sbig)


def _tc_combine_body(p_ref, o_ref):
    o_ref[...] = p_ref[0] + p_ref[1]


_PROWS = N * D // 128


def _tc_combine(parts_lin):
    return pl.pallas_call(
        _tc_combine_body,
        out_shape=jax.ShapeDtypeStruct((_PROWS, 128), jnp.float32),
    )(parts_lin)


@functools.cache
def _sc_kernels():
    mesh = plsc.VectorSubcoreMesh(core_axis_name="c", subcore_axis_name="s")
    gather = pl.kernel(
        _sc_gather_body,
        out_type=jax.ShapeDtypeStruct((NW, NCH, CH, D), jnp.float32),
        mesh=mesh,
        scratch_types=[
            pltpu.VMEM((NCH, CH), jnp.int32),
            pltpu.VMEM((NCH, CH, D), jnp.float32),
            pltpu.SemaphoreType.DMA,
        ],
        compiler_params=pltpu.CompilerParams(use_tc_tiling_on_sc=False),
    )
    scatter = pl.kernel(
        _sc_scatter_body,
        out_type=jax.ShapeDtypeStruct((NC, N, D), jnp.float32),
        mesh=mesh,
        scratch_types=[
            pltpu.VMEM((NCH, CH), jnp.int32),
            pltpu.VMEM((NCH, CH, D), jnp.float32),
            pltpu.VMEM_SHARED((N, D), jnp.float32),
            pltpu.SemaphoreType.DMA,
        ],
        compiler_params=pltpu.CompilerParams(use_tc_tiling_on_sc=False),
    )
    return gather, scatter


def kernel(node_features, edge_src, edge_dst, edge_attr, edge_scalars, W1, W2):
    _sc_gather, _sc_scatter = _sc_kernels()
    pad = E_PAD - E
    src = jnp.pad(edge_src, (0, pad)).reshape(NW, NCH, CH)
    dst = jnp.pad(edge_dst, (0, pad)).reshape(NW, NCH, CH)
    sat = jnp.pad(edge_scalars.T, ((0, 0), (0, pad)))
    apk = jnp.pad(edge_attr[:, 0], (0, pad)).reshape(E_PAD // 8, 8)

    w1t = (W1 * (_INV_SQRT3 / 256.0)).T.astype(jnp.bfloat16)
    w2b = W2.astype(jnp.bfloat16)
    m8 = np.arange(8)
    l128 = np.arange(128)
    c2048 = np.arange(8 * HID)
    arep = (m8[:, None] == l128[None, :] // D).astype(np.float32)
    rbig = (l128[:, None] == (c2048[None, :] // HID) * D + (c2048[None, :] % HID) // D
            ).astype(jnp.bfloat16)
    sbig = ((c2048[:, None] // HID == l128[None, :] // D)
            & (c2048[:, None] % D == l128[None, :] % D)).astype(jnp.bfloat16)

    x_packed = _sc_gather(node_features, src).reshape(E_PAD // 8, 128)
    ef = _tc_edge(sat, x_packed, apk, w1t, w2b, arep, rbig, sbig).reshape(
        NW, NCH, CH, D
    )
    zeros = jnp.zeros((N, D), jnp.float32)
    parts = _sc_scatter(ef, dst, zeros)
    return _tc_combine(parts.reshape(NC, _PROWS, 128)).reshape(N, D)


# speedup vs baseline: 2.1250x; 1.0124x over previous
import functools
import math

import jax
import jax.numpy as jnp
import numpy as np
from jax import lax
from jax.experimental import pallas as pl
from jax.experimental.pallas import tpu as pltpu
from jax.experimental.pallas import tpu_sc as plsc

N = 10000
E = 160000
D = 16
HID = 256

NC = 2
NS = 16
NW = NC * NS
CH = 128
NCH = 40
PER_W = NCH * CH
E_PAD = NW * PER_W
ZR = N // NS


def _sc_gather_body(nf_hbm, src_hbm, out_hbm, idx_v, rows_v, sem):
    c = lax.axis_index("c")
    s = lax.axis_index("s")
    wid = s * NC + c
    pltpu.sync_copy(src_hbm.at[wid], idx_v)
    for g in range(0, NCH, 8):
        cps = [
            pltpu.async_copy(nf_hbm.at[idx_v.at[g + b]], rows_v.at[g + b], sem)
            for b in range(8)
        ]
        for cp in cps:
            cp.wait()
    pltpu.sync_copy(rows_v, out_hbm.at[wid])


def _sc_scatter_body(ef_hbm, dst_hbm, zero_hbm, part_hbm, idx_v, ef_v, acc, sem):
    c = lax.axis_index("c")
    s = lax.axis_index("s")
    wid = s * NC + c
    pltpu.sync_copy(zero_hbm.at[pl.ds(s * ZR, ZR)], acc.at[pl.ds(s * ZR, ZR)])
    pltpu.sync_copy(dst_hbm.at[wid], idx_v)
    pltpu.sync_copy(ef_hbm.at[wid], ef_v)
    plsc.subcore_barrier()
    for g in range(0, NCH, 8):
        cps = [
            pltpu.async_copy(ef_v.at[g + b], acc.at[idx_v.at[g + b]], sem, add=True)
            for b in range(8)
        ]
        for cp in cps:
            cp.wait()
    plsc.subcore_barrier()
    pltpu.sync_copy(acc.at[pl.ds(s * ZR, ZR)], part_hbm.at[c, pl.ds(s * ZR, ZR)])


_BLK = 16384
_INV_SQRT3 = 1.0 / math.sqrt(3.0)


def _tc_edge_body(sat_ref, x_ref, apk_ref, w1t_ref, w2_ref, arep_ref, rbig_ref, sbig_ref, out_ref):
    z_t = jnp.dot(
        w1t_ref[...], sat_ref[...].astype(jnp.bfloat16),
        preferred_element_type=jnp.float32,
    )
    h_t = jnp.maximum(z_t, 0.0).astype(jnp.bfloat16)
    w = lax.dot_general(
        h_t, w2_ref[...], (((0,), (0,)), ((), ())),
        preferred_element_type=jnp.float32,
    )
    wre = w.reshape(_BLK // 8, 8 * HID)
    attr_rep = jnp.dot(apk_ref[...], arep_ref[...], preferred_element_type=jnp.float32)
    xa = (x_ref[...] * attr_rep).astype(jnp.bfloat16)
    xr = jnp.dot(xa, rbig_ref[...], preferred_element_type=jnp.float32)
    prod = (xr * wre).astype(jnp.bfloat16)
    out_ref[...] = jnp.dot(prod, sbig_ref[...], preferred_element_type=jnp.float32)


def _tc_edge(sat, x_packed, apk, w1t, w2, arep, rbig, sbig):
    return pl.pallas_call(
        _tc_edge_body,
        grid=(E_PAD // _BLK,),
        in_specs=[
            pl.BlockSpec((3, _BLK), lambda i: (0, i)),
            pl.BlockSpec((_BLK // 8, 128), lambda i: (i, 0)),
            pl.BlockSpec((_BLK // 8, 8), lambda i: (i, 0)),
            pl.BlockSpec((HID, 3), lambda i: (0, 0)),
            pl.BlockSpec((HID, HID), lambda i: (0, 0)),
            pl.BlockSpec((8, 128), lambda i: (0, 0)),
            pl.BlockSpec((128, 8 * HID), lambda i: (0, 0)),
            pl.BlockSpec((8 * HID, 128), lambda i: (0, 0)),
        ],
        out_specs=pl.BlockSpec((_BLK // 8, 128), lambda i: (i, 0)),
        out_shape=jax.ShapeDtypeStruct((E_PAD // 8, 128), jnp.float32),
    )(sat, x_packed, apk, w1t, w2, arep, rbig, sbig)


def _tc_combine_body(p_ref, o_ref):
    o_ref[...] = p_ref[0] + p_ref[1]


_PROWS = N * D // 128


def _tc_combine(parts_lin):
    return pl.pallas_call(
        _tc_combine_body,
        out_shape=jax.ShapeDtypeStruct((_PROWS, 128), jnp.float32),
    )(parts_lin)


@functools.cache
def _sc_kernels():
    mesh = plsc.VectorSubcoreMesh(core_axis_name="c", subcore_axis_name="s")
    gather = pl.kernel(
        _sc_gather_body,
        out_type=jax.ShapeDtypeStruct((NW, NCH, CH, D), jnp.float32),
        mesh=mesh,
        scratch_types=[
            pltpu.VMEM((NCH, CH), jnp.int32),
            pltpu.VMEM((NCH, CH, D), jnp.float32),
            pltpu.SemaphoreType.DMA,
        ],
        compiler_params=pltpu.CompilerParams(use_tc_tiling_on_sc=False),
    )
    scatter = pl.kernel(
        _sc_scatter_body,
        out_type=jax.ShapeDtypeStruct((NC, N, D), jnp.float32),
        mesh=mesh,
        scratch_types=[
            pltpu.VMEM((NCH, CH), jnp.int32),
            pltpu.VMEM((NCH, CH, D), jnp.float32),
            pltpu.VMEM_SHARED((N, D), jnp.float32),
            pltpu.SemaphoreType.DMA,
        ],
        compiler_params=pltpu.CompilerParams(use_tc_tiling_on_sc=False),
    )
    return gather, scatter


def kernel(node_features, edge_src, edge_dst, edge_attr, edge_scalars, W1, W2):
    _sc_gather, _sc_scatter = _sc_kernels()
    pad = E_PAD - E
    src = jnp.pad(edge_src, (0, pad)).reshape(NW, NCH, CH)
    dst = jnp.pad(edge_dst, (0, pad)).reshape(NW, NCH, CH)
    sat = jnp.pad(edge_scalars.T, ((0, 0), (0, pad)))
    apk = jnp.pad(edge_attr[:, 0], (0, pad)).reshape(E_PAD // 8, 8)

    w1t = (W1 * (_INV_SQRT3 / 256.0)).T.astype(jnp.bfloat16)
    w2b = W2.astype(jnp.bfloat16)
    m8 = np.arange(8)
    l128 = np.arange(128)
    c2048 = np.arange(8 * HID)
    arep = (m8[:, None] == l128[None, :] // D).astype(np.float32)
    rbig = (l128[:, None] == (c2048[None, :] // HID) * D + (c2048[None, :] % HID) // D
            ).astype(jnp.bfloat16)
    sbig = ((c2048[:, None] // HID == l128[None, :] // D)
            & (c2048[:, None] % D == l128[None, :] % D)).astype(jnp.bfloat16)

    x_packed = _sc_gather(node_features, src).reshape(E_PAD // 8, 128)
    ef = _tc_edge(sat, x_packed, apk, w1t, w2b, arep, rbig, sbig).reshape(
        NW, NCH, CH, D
    )
    zeros = jnp.zeros((N, D), jnp.float32)
    parts = _sc_scatter(ef, dst, zeros)
    return _tc_combine(parts.reshape(NC, _PROWS, 128)).reshape(N, D)
